# baseline (device time: 317278 ns/iter reference)
import jax
import jax.numpy as jnp
from jax import lax
from jax.experimental import pallas as pl
from jax.experimental.pallas import tpu as pltpu

SCALE = 64 ** -0.5


def _partial_body(q_ref, k_ref, v_ref, mask_ref, onum_ref, m_ref, l_ref):
    skv, h, d = k_ref.shape[1:]
    q = q_ref[0] * SCALE
    k = k_ref[0].reshape(skv * h, d)
    v = v_ref[0].reshape(skv * h, d)
    s = lax.dot_general(
        q, k, (((1,), (1,)), ((), ())),
        preferred_element_type=jnp.float32,
    )
    m = jnp.max(s, axis=1, keepdims=True)
    p = jnp.exp(s - m) * mask_ref[...]
    l = jnp.sum(p, axis=1, keepdims=True)
    o = lax.dot_general(
        p, v, (((1,), (0,)), ((), ())),
        preferred_element_type=jnp.float32,
    )
    onum_ref[0] = o
    m_ref[0] = m
    l_ref[0] = l


def _merge_body(onum_ref, stats_ref, out_ref,
                r_onum, r_stats, send_sems, recv_sems):
    my_x = lax.axis_index("x")
    my_y = lax.axis_index("y")
    my_z = lax.axis_index("z")
    partner = (my_x, my_y, 1 - my_z)

    copies = []
    for i, (src, dst) in enumerate(((onum_ref, r_onum), (stats_ref, r_stats))):
        rdma = pltpu.make_async_remote_copy(
            src_ref=src,
            dst_ref=dst,
            send_sem=send_sems.at[i],
            recv_sem=recv_sems.at[i],
            device_id=partner,
            device_id_type=pl.DeviceIdType.MESH,
        )
        rdma.start()
        copies.append(rdma)
    for rdma in copies:
        rdma.wait()

    m0 = stats_ref[:, 0]
    l0 = stats_ref[:, 1]
    m1 = r_stats[:, 0]
    l1 = r_stats[:, 1]
    mg = jnp.maximum(m0, m1)
    a0 = jnp.exp(m0 - mg)
    a1 = jnp.exp(m1 - mg)
    lg = a0 * l0 + a1 * l1
    out_ref[...] = (onum_ref[...] * a0 + r_onum[...] * a1) / lg


def kernel(Q, K, V):
    b, sq, h, d = Q.shape
    skv = K.shape[1]
    kh = skv * h

    q3 = Q.reshape(b, h, d)

    cols = jax.lax.broadcasted_iota(jnp.int32, (h, kh), 1)
    rows = jax.lax.broadcasted_iota(jnp.int32, (h, kh), 0)
    mask = (cols % h == rows).astype(jnp.float32)

    onum, m, l = pl.pallas_call(
        _partial_body,
        grid=(b,),
        in_specs=[
            pl.BlockSpec((1, h, d), lambda i: (i, 0, 0)),
            pl.BlockSpec((1, skv, h, d), lambda i: (i, 0, 0, 0)),
            pl.BlockSpec((1, skv, h, d), lambda i: (i, 0, 0, 0)),
            pl.BlockSpec((h, kh), lambda i: (0, 0)),
        ],
        out_specs=[
            pl.BlockSpec((1, h, d), lambda i: (i, 0, 0)),
            pl.BlockSpec((1, h, 1), lambda i: (i, 0, 0)),
            pl.BlockSpec((1, h, 1), lambda i: (i, 0, 0)),
        ],
        out_shape=[
            jax.ShapeDtypeStruct((b, h, d), jnp.float32),
            jax.ShapeDtypeStruct((b, h, 1), jnp.float32),
            jax.ShapeDtypeStruct((b, h, 1), jnp.float32),
        ],
        compiler_params=pltpu.CompilerParams(
            vmem_limit_bytes=96 * 1024 * 1024,
        ),
    )(q3, K, V, mask)

    stats = jnp.stack([m, l], axis=1)

    out = pl.pallas_call(
        _merge_body,
        in_specs=[
            pl.BlockSpec(memory_space=pltpu.VMEM),
            pl.BlockSpec(memory_space=pltpu.VMEM),
        ],
        out_specs=pl.BlockSpec(memory_space=pltpu.VMEM),
        out_shape=jax.ShapeDtypeStruct((b, h, d), jnp.float32),
        scratch_shapes=[
            pltpu.VMEM((b, h, d), jnp.float32),
            pltpu.VMEM((b, 2, h, 1), jnp.float32),
            pltpu.SemaphoreType.DMA((2,)),
            pltpu.SemaphoreType.DMA((2,)),
        ],
        compiler_params=pltpu.CompilerParams(has_side_effects=True),
    )(onum, stats)
    return out.reshape(b, sq, h, d)


# device time: 67277 ns/iter; 4.7160x vs baseline; 4.7160x over previous
import jax
import jax.numpy as jnp
from jax import lax
from jax.experimental import pallas as pl
from jax.experimental.pallas import tpu as pltpu

SCALE = 64 ** -0.5


def _partial_body(q_ref, kt_ref, vt_ref, onum_ref, m_ref, l_ref):
    h = q_ref.shape[1]
    q = q_ref[0] * SCALE
    for i in range(h):
        qh = q[i:i + 1, :]
        kh = kt_ref[0, i]
        s = lax.dot_general(
            qh, kh, (((1,), (0,)), ((), ())),
            preferred_element_type=jnp.float32,
        )
        m = jnp.max(s, axis=1, keepdims=True)
        p = jnp.exp(s - m)
        l = jnp.sum(p, axis=1, keepdims=True)
        vh = vt_ref[0, i]
        o = lax.dot_general(
            p, vh, (((1,), (1,)), ((), ())),
            preferred_element_type=jnp.float32,
        )
        onum_ref[0, i:i + 1, :] = o
        m_ref[0, i:i + 1, :] = m
        l_ref[0, i:i + 1, :] = l


def _merge_body(onum_ref, stats_ref, out_ref,
                r_onum, r_stats, send_sems, recv_sems):
    my_x = lax.axis_index("x")
    my_y = lax.axis_index("y")
    my_z = lax.axis_index("z")
    partner = (my_x, my_y, 1 - my_z)

    copies = []
    for i, (src, dst) in enumerate(((onum_ref, r_onum), (stats_ref, r_stats))):
        rdma = pltpu.make_async_remote_copy(
            src_ref=src,
            dst_ref=dst,
            send_sem=send_sems.at[i],
            recv_sem=recv_sems.at[i],
            device_id=partner,
            device_id_type=pl.DeviceIdType.MESH,
        )
        rdma.start()
        copies.append(rdma)
    for rdma in copies:
        rdma.wait()

    m0 = stats_ref[:, 0]
    l0 = stats_ref[:, 1]
    m1 = r_stats[:, 0]
    l1 = r_stats[:, 1]
    mg = jnp.maximum(m0, m1)
    a0 = jnp.exp(m0 - mg)
    a1 = jnp.exp(m1 - mg)
    lg = a0 * l0 + a1 * l1
    out_ref[...] = (onum_ref[...] * a0 + r_onum[...] * a1) / lg


def kernel(Q, K, V):
    b, sq, h, d = Q.shape
    skv = K.shape[1]

    q3 = Q.reshape(b, h, d)
    kt = jnp.transpose(K, (0, 2, 3, 1))
    vt = jnp.transpose(V, (0, 2, 3, 1))

    onum, m, l = pl.pallas_call(
        _partial_body,
        grid=(b,),
        in_specs=[
            pl.BlockSpec((1, h, d), lambda i: (i, 0, 0)),
            pl.BlockSpec((1, h, d, skv), lambda i: (i, 0, 0, 0)),
            pl.BlockSpec((1, h, d, skv), lambda i: (i, 0, 0, 0)),
        ],
        out_specs=[
            pl.BlockSpec((1, h, d), lambda i: (i, 0, 0)),
            pl.BlockSpec((1, h, 1), lambda i: (i, 0, 0)),
            pl.BlockSpec((1, h, 1), lambda i: (i, 0, 0)),
        ],
        out_shape=[
            jax.ShapeDtypeStruct((b, h, d), jnp.float32),
            jax.ShapeDtypeStruct((b, h, 1), jnp.float32),
            jax.ShapeDtypeStruct((b, h, 1), jnp.float32),
        ],
        compiler_params=pltpu.CompilerParams(
            vmem_limit_bytes=96 * 1024 * 1024,
        ),
    )(q3, kt, vt)

    stats = jnp.stack([m, l], axis=1)

    out = pl.pallas_call(
        _merge_body,
        in_specs=[
            pl.BlockSpec(memory_space=pltpu.VMEM),
            pl.BlockSpec(memory_space=pltpu.VMEM),
        ],
        out_specs=pl.BlockSpec(memory_space=pltpu.VMEM),
        out_shape=jax.ShapeDtypeStruct((b, h, d), jnp.float32),
        scratch_shapes=[
            pltpu.VMEM((b, h, d), jnp.float32),
            pltpu.VMEM((b, 2, h, 1), jnp.float32),
            pltpu.SemaphoreType.DMA((2,)),
            pltpu.SemaphoreType.DMA((2,)),
        ],
        compiler_params=pltpu.CompilerParams(has_side_effects=True),
    )(onum, stats)
    return out.reshape(b, sq, h, d)


# device time: 33703 ns/iter; 9.4139x vs baseline; 1.9962x over previous
import jax
import jax.numpy as jnp
from jax import lax
from jax.experimental import pallas as pl
from jax.experimental.pallas import tpu as pltpu

SCALE = 64 ** -0.5
BQ = 4


def _partial_body(qidx_ref, q_ref, kt_ref, vt_ref, onum_ref, m_ref, l_ref):
    h = q_ref.shape[1]
    q = q_ref[0] * SCALE
    for i in range(h):
        qh = q[i:i + 1, :]
        kh = kt_ref[0, i]
        s = lax.dot_general(
            qh, kh, (((1,), (0,)), ((), ())),
            preferred_element_type=jnp.float32,
        )
        m = jnp.max(s, axis=1, keepdims=True)
        p = jnp.exp(s - m)
        l = jnp.sum(p, axis=1, keepdims=True)
        vh = vt_ref[0, i]
        o = lax.dot_general(
            p, vh, (((1,), (1,)), ((), ())),
            preferred_element_type=jnp.float32,
        )
        onum_ref[0, i:i + 1, :] = o
        m_ref[0, i:i + 1, :] = m
        l_ref[0, i:i + 1, :] = l


def _merge_body(onum_ref, stats_ref, out_ref,
                r_onum, r_stats, send_sems, recv_sems):
    my_x = lax.axis_index("x")
    my_y = lax.axis_index("y")
    my_z = lax.axis_index("z")
    qidx = my_x * 2 + my_y

    copies = []
    for i, (src, dst) in enumerate(((onum_ref, r_onum), (stats_ref, r_stats))):
        rdma = pltpu.make_async_remote_copy(
            src_ref=src,
            dst_ref=dst,
            send_sem=send_sems.at[i],
            recv_sem=recv_sems.at[i],
            device_id=(my_x, my_y, 1 - my_z),
            device_id_type=pl.DeviceIdType.MESH,
        )
        rdma.start()
        copies.append(rdma)
    for rdma in copies:
        rdma.wait()

    m0 = stats_ref[:, 0]
    l0 = stats_ref[:, 1]
    m1 = r_stats[:, 0]
    l1 = r_stats[:, 1]
    mg = jnp.maximum(m0, m1)
    a0 = jnp.exp(m0 - mg)
    a1 = jnp.exp(m1 - mg)
    lg = a0 * l0 + a1 * l1
    merged = (onum_ref[...] * a0 + r_onum[...] * a1) / lg
    out_ref[pl.ds(qidx * BQ, BQ)] = merged

    rdma_y = pltpu.make_async_remote_copy(
        src_ref=out_ref.at[pl.ds(qidx * BQ, BQ)],
        dst_ref=out_ref.at[pl.ds(qidx * BQ, BQ)],
        send_sem=send_sems.at[2],
        recv_sem=recv_sems.at[2],
        device_id=(my_x, 1 - my_y, my_z),
        device_id_type=pl.DeviceIdType.MESH,
    )
    rdma_y.start()
    rdma_y.wait()

    rdma_x = pltpu.make_async_remote_copy(
        src_ref=out_ref.at[pl.ds(my_x * 2 * BQ, 2 * BQ)],
        dst_ref=out_ref.at[pl.ds(my_x * 2 * BQ, 2 * BQ)],
        send_sem=send_sems.at[3],
        recv_sem=recv_sems.at[3],
        device_id=(1 - my_x, my_y, my_z),
        device_id_type=pl.DeviceIdType.MESH,
    )
    rdma_x.start()
    rdma_x.wait()


def kernel(Q, K, V):
    b, sq, h, d = Q.shape
    skv = K.shape[1]

    q3 = Q.reshape(b, h, d)
    kt = jnp.transpose(K, (0, 2, 3, 1))
    vt = jnp.transpose(V, (0, 2, 3, 1))

    qidx = (lax.axis_index("x") * 2 + lax.axis_index("y")).astype(jnp.int32)
    qidx_arr = qidx.reshape(1)

    onum, m, l = pl.pallas_call(
        _partial_body,
        grid_spec=pltpu.PrefetchScalarGridSpec(
            num_scalar_prefetch=1,
            grid=(BQ,),
            in_specs=[
                pl.BlockSpec((1, h, d), lambda i, s: (s[0] * BQ + i, 0, 0)),
                pl.BlockSpec((1, h, d, skv),
                             lambda i, s: (s[0] * BQ + i, 0, 0, 0)),
                pl.BlockSpec((1, h, d, skv),
                             lambda i, s: (s[0] * BQ + i, 0, 0, 0)),
            ],
            out_specs=[
                pl.BlockSpec((1, h, d), lambda i, s: (i, 0, 0)),
                pl.BlockSpec((1, h, 1), lambda i, s: (i, 0, 0)),
                pl.BlockSpec((1, h, 1), lambda i, s: (i, 0, 0)),
            ],
        ),
        out_shape=[
            jax.ShapeDtypeStruct((BQ, h, d), jnp.float32),
            jax.ShapeDtypeStruct((BQ, h, 1), jnp.float32),
            jax.ShapeDtypeStruct((BQ, h, 1), jnp.float32),
        ],
        compiler_params=pltpu.CompilerParams(
            vmem_limit_bytes=96 * 1024 * 1024,
        ),
    )(qidx_arr, q3, kt, vt)

    stats = jnp.stack([m, l], axis=1)

    out = pl.pallas_call(
        _merge_body,
        in_specs=[
            pl.BlockSpec(memory_space=pltpu.VMEM),
            pl.BlockSpec(memory_space=pltpu.VMEM),
        ],
        out_specs=pl.BlockSpec(memory_space=pltpu.VMEM),
        out_shape=jax.ShapeDtypeStruct((b, h, d), jnp.float32),
        scratch_shapes=[
            pltpu.VMEM((BQ, h, d), jnp.float32),
            pltpu.VMEM((BQ, 2, h, 1), jnp.float32),
            pltpu.SemaphoreType.DMA((4,)),
            pltpu.SemaphoreType.DMA((4,)),
        ],
        compiler_params=pltpu.CompilerParams(has_side_effects=True),
    )(onum, stats)
    return out.reshape(b, sq, h, d)


# device time: 30298 ns/iter; 10.4719x vs baseline; 1.1124x over previous
import jax
import jax.numpy as jnp
from jax import lax
from jax.experimental import pallas as pl
from jax.experimental.pallas import tpu as pltpu

SCALE = 64 ** -0.5
BQ = 4


def _body(qidx_ref, q_ref, kt_ref, vt_ref, out_ref,
          onum_s, stats_s, r_onum, r_stats,
          zsend, zrecv, gsend, grecv):
    h, d = q_ref.shape[1:]
    i = pl.program_id(0)
    q = q_ref[0] * SCALE
    for hh in range(h):
        qh = q[hh:hh + 1, :]
        kh = kt_ref[0, hh]
        s = lax.dot_general(
            qh, kh, (((1,), (0,)), ((), ())),
            preferred_element_type=jnp.float32,
        )
        m = jnp.max(s, axis=1, keepdims=True)
        p = jnp.exp(s - m)
        l = jnp.sum(p, axis=1, keepdims=True)
        vh = vt_ref[0, hh]
        o = lax.dot_general(
            p, vh, (((1,), (1,)), ((), ())),
            preferred_element_type=jnp.float32,
        )
        onum_s[i, hh:hh + 1, :] = o
        stats_s[i, 0, hh:hh + 1, :] = m
        stats_s[i, 1, hh:hh + 1, :] = l

    @pl.when(i == BQ - 1)
    def _finalize():
        my_x = lax.axis_index("x")
        my_y = lax.axis_index("y")
        my_z = lax.axis_index("z")
        qidx = my_x * 2 + my_y
        z_peer = (my_x, my_y, 1 - my_z)
        g_peers = (
            (my_x, 1 - my_y, my_z),
            (1 - my_x, my_y, my_z),
            (1 - my_x, 1 - my_y, my_z),
        )
        g_quarters = (
            my_x * 2 + (1 - my_y),
            (1 - my_x) * 2 + my_y,
            (1 - my_x) * 2 + (1 - my_y),
        )


        zcopies = []
        for j, (src, dst) in enumerate(((onum_s, r_onum),
                                        (stats_s, r_stats))):
            rdma = pltpu.make_async_remote_copy(
                src_ref=src, dst_ref=dst,
                send_sem=zsend.at[j], recv_sem=zrecv.at[j],
                device_id=z_peer, device_id_type=pl.DeviceIdType.MESH,
            )
            rdma.start()
            zcopies.append(rdma)
        for rdma in zcopies:
            rdma.wait()

        m0 = stats_s[:, 0]
        l0 = stats_s[:, 1]
        m1 = r_stats[:, 0]
        l1 = r_stats[:, 1]
        mg = jnp.maximum(m0, m1)
        a0 = jnp.exp(m0 - mg)
        a1 = jnp.exp(m1 - mg)
        lg = a0 * l0 + a1 * l1
        merged = (onum_s[...] * a0 + r_onum[...] * a1) / lg
        out_ref[pl.ds(qidx * BQ, BQ)] = merged

        sends = []
        for j, peer in enumerate(g_peers):
            rdma = pltpu.make_async_remote_copy(
                src_ref=out_ref.at[pl.ds(qidx * BQ, BQ)],
                dst_ref=out_ref.at[pl.ds(qidx * BQ, BQ)],
                send_sem=gsend.at[j], recv_sem=grecv.at[j],
                device_id=peer, device_id_type=pl.DeviceIdType.MESH,
            )
            rdma.start()
            sends.append(rdma)
        for j, qq in enumerate(g_quarters):
            rdma = pltpu.make_async_remote_copy(
                src_ref=out_ref.at[pl.ds(qq * BQ, BQ)],
                dst_ref=out_ref.at[pl.ds(qq * BQ, BQ)],
                send_sem=gsend.at[j], recv_sem=grecv.at[j],
                device_id=g_peers[j], device_id_type=pl.DeviceIdType.MESH,
            )
            rdma.wait_recv()
        for rdma in sends:
            rdma.wait_send()


def kernel(Q, K, V):
    b, sq, h, d = Q.shape
    skv = K.shape[1]

    q3 = Q.reshape(b, h, d)
    kt = jnp.transpose(K, (0, 2, 3, 1))
    vt = jnp.transpose(V, (0, 2, 3, 1))

    qidx = (lax.axis_index("x") * 2 + lax.axis_index("y")).astype(jnp.int32)
    qidx_arr = qidx.reshape(1)

    out = pl.pallas_call(
        _body,
        grid_spec=pltpu.PrefetchScalarGridSpec(
            num_scalar_prefetch=1,
            grid=(BQ,),
            in_specs=[
                pl.BlockSpec((1, h, d), lambda i, s: (s[0] * BQ + i, 0, 0)),
                pl.BlockSpec((1, h, d, skv),
                             lambda i, s: (s[0] * BQ + i, 0, 0, 0)),
                pl.BlockSpec((1, h, d, skv),
                             lambda i, s: (s[0] * BQ + i, 0, 0, 0)),
            ],
            out_specs=pl.BlockSpec((b, h, d), lambda i, s: (0, 0, 0)),
            scratch_shapes=[
                pltpu.VMEM((BQ, h, d), jnp.float32),
                pltpu.VMEM((BQ, 2, h, 1), jnp.float32),
                pltpu.VMEM((BQ, h, d), jnp.float32),
                pltpu.VMEM((BQ, 2, h, 1), jnp.float32),
                pltpu.SemaphoreType.DMA((2,)),
                pltpu.SemaphoreType.DMA((2,)),
                pltpu.SemaphoreType.DMA((3,)),
                pltpu.SemaphoreType.DMA((3,)),
            ],
        ),
        out_shape=jax.ShapeDtypeStruct((b, h, d), jnp.float32),
        compiler_params=pltpu.CompilerParams(
            vmem_limit_bytes=96 * 1024 * 1024,
            has_side_effects=True,
        ),
    )(qidx_arr, q3, kt, vt)
    return out.reshape(b, sq, h, d)


# device time: 28508 ns/iter; 11.1294x vs baseline; 1.0628x over previous
import jax
import jax.numpy as jnp
from jax import lax
from jax.experimental import pallas as pl
from jax.experimental.pallas import tpu as pltpu

SCALE = 64 ** -0.5
BQ = 4


def _body(qidx_ref, q_ref, kt_ref, vt_ref, out_ref,
          onum_s, stats_s, r_onum, r_stats,
          zsend, zrecv, gsend, grecv):
    _, h, d, skv = kt_ref.shape
    hd = h * d
    i = pl.program_id(0)

    rows = lax.broadcasted_iota(jnp.int32, (h, hd), 0)
    cols = lax.broadcasted_iota(jnp.int32, (h, hd), 1)
    e_mat = (cols // d == rows).astype(jnp.float32)

    q_row = q_ref[0] * SCALE
    w = q_row * e_mat
    k2 = kt_ref[0].reshape(hd, skv)
    v2 = vt_ref[0].reshape(hd, skv)
    s = lax.dot_general(
        w, k2, (((1,), (0,)), ((), ())),
        preferred_element_type=jnp.float32,
    )
    m = jnp.max(s, axis=1, keepdims=True)
    p = jnp.exp(s - m)
    l = jnp.sum(p, axis=1, keepdims=True)
    o_full = lax.dot_general(
        p, v2, (((1,), (1,)), ((), ())),
        preferred_element_type=jnp.float32,
    )
    o_flat = jnp.sum(o_full * e_mat, axis=0, keepdims=True)
    for jj in range(BQ):
        @pl.when(i == jj)
        def _store(jj=jj):
            onum_s[jj:jj + 1, :] = o_flat
            stats_s[0, :, jj:jj + 1] = m
            stats_s[1, :, jj:jj + 1] = l

    @pl.when(i == BQ - 1)
    def _finalize():
        my_x = lax.axis_index("x")
        my_y = lax.axis_index("y")
        my_z = lax.axis_index("z")
        qidx = my_x * 2 + my_y
        z_peer = (my_x, my_y, 1 - my_z)
        g_peers = (
            (my_x, 1 - my_y, my_z),
            (1 - my_x, my_y, my_z),
            (1 - my_x, 1 - my_y, my_z),
        )
        g_quarters = (
            my_x * 2 + (1 - my_y),
            (1 - my_x) * 2 + my_y,
            (1 - my_x) * 2 + (1 - my_y),
        )

        zcopies = []
        for j, (src, dst) in enumerate(((onum_s, r_onum),
                                        (stats_s, r_stats))):
            rdma = pltpu.make_async_remote_copy(
                src_ref=src, dst_ref=dst,
                send_sem=zsend.at[j], recv_sem=zrecv.at[j],
                device_id=z_peer, device_id_type=pl.DeviceIdType.MESH,
            )
            rdma.start()
            zcopies.append(rdma)
        for rdma in zcopies:
            rdma.wait()

        m0 = stats_s[0]
        l0 = stats_s[1]
        m1 = r_stats[0]
        l1 = r_stats[1]
        mg = jnp.maximum(m0, m1)
        a0 = jnp.exp(m0 - mg)
        a1 = jnp.exp(m1 - mg)
        lg = a0 * l0 + a1 * l1
        w0t = a0 / lg
        w1t = a1 / lg
        w0 = lax.dot_general(
            w0t, e_mat, (((0,), (0,)), ((), ())),
            preferred_element_type=jnp.float32,
        )
        w1 = lax.dot_general(
            w1t, e_mat, (((0,), (0,)), ((), ())),
            preferred_element_type=jnp.float32,
        )
        merged = onum_s[...] * w0 + r_onum[...] * w1
        for qq in range(4):
            @pl.when(qidx == qq)
            def _store_out(qq=qq):
                out_ref[qq * BQ:(qq + 1) * BQ, :] = merged

        sends = []
        for j, peer in enumerate(g_peers):
            rdma = pltpu.make_async_remote_copy(
                src_ref=out_ref.at[pl.ds(qidx * BQ, BQ)],
                dst_ref=out_ref.at[pl.ds(qidx * BQ, BQ)],
                send_sem=gsend.at[j], recv_sem=grecv.at[j],
                device_id=peer, device_id_type=pl.DeviceIdType.MESH,
            )
            rdma.start()
            sends.append(rdma)
        for j, qq in enumerate(g_quarters):
            rdma = pltpu.make_async_remote_copy(
                src_ref=out_ref.at[pl.ds(qq * BQ, BQ)],
                dst_ref=out_ref.at[pl.ds(qq * BQ, BQ)],
                send_sem=gsend.at[j], recv_sem=grecv.at[j],
                device_id=g_peers[j], device_id_type=pl.DeviceIdType.MESH,
            )
            rdma.wait_recv()
        for rdma in sends:
            rdma.wait_send()


def kernel(Q, K, V):
    b, sq, h, d = Q.shape
    skv = K.shape[1]
    hd = h * d

    q2 = Q.reshape(b, 1, hd)
    kt = jnp.transpose(K, (0, 2, 3, 1))
    vt = jnp.transpose(V, (0, 2, 3, 1))

    qidx = (lax.axis_index("x") * 2 + lax.axis_index("y")).astype(jnp.int32)
    qidx_arr = qidx.reshape(1)

    out = pl.pallas_call(
        _body,
        grid_spec=pltpu.PrefetchScalarGridSpec(
            num_scalar_prefetch=1,
            grid=(BQ,),
            in_specs=[
                pl.BlockSpec((1, 1, hd), lambda i, s: (s[0] * BQ + i, 0, 0)),
                pl.BlockSpec((1, h, d, skv),
                             lambda i, s: (s[0] * BQ + i, 0, 0, 0)),
                pl.BlockSpec((1, h, d, skv),
                             lambda i, s: (s[0] * BQ + i, 0, 0, 0)),
            ],
            out_specs=pl.BlockSpec((b, hd), lambda i, s: (0, 0)),
            scratch_shapes=[
                pltpu.VMEM((BQ, hd), jnp.float32),
                pltpu.VMEM((2, h, BQ), jnp.float32),
                pltpu.VMEM((BQ, hd), jnp.float32),
                pltpu.VMEM((2, h, BQ), jnp.float32),
                pltpu.SemaphoreType.DMA((2,)),
                pltpu.SemaphoreType.DMA((2,)),
                pltpu.SemaphoreType.DMA((3,)),
                pltpu.SemaphoreType.DMA((3,)),
            ],
        ),
        out_shape=jax.ShapeDtypeStruct((b, hd), jnp.float32),
        compiler_params=pltpu.CompilerParams(
            vmem_limit_bytes=96 * 1024 * 1024,
            has_side_effects=True,
        ),
    )(qidx_arr, q2, kt, vt)
    return out.reshape(b, sq, h, d)


# device time: 23937 ns/iter; 13.2547x vs baseline; 1.1910x over previous
import jax
import jax.numpy as jnp
from jax import lax
from jax.experimental import pallas as pl
from jax.experimental.pallas import tpu as pltpu

SCALE = 64 ** -0.5
BQ = 4


def _body(qidx_ref, q_ref, kt_ref, vt_ref, out_ref,
          onum_s, stats_s, r_onum, r_stats,
          zsend, zrecv, gsend, grecv):
    _, h, d, skv = kt_ref.shape
    hd = h * d
    i = pl.program_id(0)

    my_x = lax.axis_index("x")
    my_y = lax.axis_index("y")
    my_z = lax.axis_index("z")
    peers4 = (
        (my_x, my_y, 1 - my_z),
        (my_x, 1 - my_y, my_z),
        (1 - my_x, my_y, my_z),
        (1 - my_x, 1 - my_y, my_z),
    )
    barrier = pltpu.get_barrier_semaphore()

    @pl.when(i == 0)
    def _barrier_signal():
        for peer in peers4:
            pl.semaphore_signal(
                barrier, inc=1, device_id=peer,
                device_id_type=pl.DeviceIdType.MESH,
            )

    rows = lax.broadcasted_iota(jnp.int32, (h, hd), 0)
    cols = lax.broadcasted_iota(jnp.int32, (h, hd), 1)
    e_mat = (cols // d == rows).astype(jnp.float32)

    q_row = q_ref[0] * SCALE
    w = q_row * e_mat
    k2 = kt_ref[0].reshape(hd, skv)
    v2 = vt_ref[0].reshape(hd, skv)
    s = lax.dot_general(
        w, k2, (((1,), (0,)), ((), ())),
        preferred_element_type=jnp.float32,
    )
    m = jnp.max(s, axis=1, keepdims=True)
    p = jnp.exp(s - m)
    l = jnp.sum(p, axis=1, keepdims=True)
    o_full = lax.dot_general(
        p, v2, (((1,), (1,)), ((), ())),
        preferred_element_type=jnp.float32,
    )
    o_flat = jnp.sum(o_full * e_mat, axis=0, keepdims=True)
    for jj in range(BQ):
        @pl.when(i == jj)
        def _store(jj=jj):
            onum_s[jj:jj + 1, :] = o_flat
            stats_s[0, :, jj:jj + 1] = m
            stats_s[1, :, jj:jj + 1] = l

    @pl.when(i == BQ - 1)
    def _finalize():
        qidx = my_x * 2 + my_y
        z_peer = peers4[0]
        g_peers = peers4[1:]
        g_quarters = (
            my_x * 2 + (1 - my_y),
            (1 - my_x) * 2 + my_y,
            (1 - my_x) * 2 + (1 - my_y),
        )

        pl.semaphore_wait(barrier, 4)

        zcopies = []
        for j, (src, dst) in enumerate(((onum_s, r_onum),
                                        (stats_s, r_stats))):
            rdma = pltpu.make_async_remote_copy(
                src_ref=src, dst_ref=dst,
                send_sem=zsend.at[j], recv_sem=zrecv.at[j],
                device_id=z_peer, device_id_type=pl.DeviceIdType.MESH,
            )
            rdma.start()
            zcopies.append(rdma)
        for rdma in zcopies:
            rdma.wait()

        m0 = stats_s[0]
        l0 = stats_s[1]
        m1 = r_stats[0]
        l1 = r_stats[1]
        mg = jnp.maximum(m0, m1)
        a0 = jnp.exp(m0 - mg)
        a1 = jnp.exp(m1 - mg)
        lg = a0 * l0 + a1 * l1
        w0t = a0 / lg
        w1t = a1 / lg
        w0 = lax.dot_general(
            w0t, e_mat, (((0,), (0,)), ((), ())),
            preferred_element_type=jnp.float32,
        )
        w1 = lax.dot_general(
            w1t, e_mat, (((0,), (0,)), ((), ())),
            preferred_element_type=jnp.float32,
        )
        merged = onum_s[...] * w0 + r_onum[...] * w1
        for qq in range(4):
            @pl.when(qidx == qq)
            def _store_out(qq=qq):
                out_ref[qq * BQ:(qq + 1) * BQ, :] = merged

        sends = []
        for j, peer in enumerate(g_peers):
            rdma = pltpu.make_async_remote_copy(
                src_ref=out_ref.at[pl.ds(qidx * BQ, BQ)],
                dst_ref=out_ref.at[pl.ds(qidx * BQ, BQ)],
                send_sem=gsend.at[j], recv_sem=grecv.at[j],
                device_id=peer, device_id_type=pl.DeviceIdType.MESH,
            )
            rdma.start()
            sends.append(rdma)
        for j, qq in enumerate(g_quarters):
            rdma = pltpu.make_async_remote_copy(
                src_ref=out_ref.at[pl.ds(qq * BQ, BQ)],
                dst_ref=out_ref.at[pl.ds(qq * BQ, BQ)],
                send_sem=gsend.at[j], recv_sem=grecv.at[j],
                device_id=g_peers[j], device_id_type=pl.DeviceIdType.MESH,
            )
            rdma.wait_recv()
        for rdma in sends:
            rdma.wait_send()


def kernel(Q, K, V):
    b, sq, h, d = Q.shape
    skv = K.shape[1]
    hd = h * d

    q2 = Q.reshape(b, 1, hd)
    kt = jnp.transpose(K, (0, 2, 3, 1))
    vt = jnp.transpose(V, (0, 2, 3, 1))

    qidx = (lax.axis_index("x") * 2 + lax.axis_index("y")).astype(jnp.int32)
    qidx_arr = qidx.reshape(1)

    out = pl.pallas_call(
        _body,
        grid_spec=pltpu.PrefetchScalarGridSpec(
            num_scalar_prefetch=1,
            grid=(BQ,),
            in_specs=[
                pl.BlockSpec((1, 1, hd), lambda i, s: (s[0] * BQ + i, 0, 0)),
                pl.BlockSpec((1, h, d, skv),
                             lambda i, s: (s[0] * BQ + i, 0, 0, 0)),
                pl.BlockSpec((1, h, d, skv),
                             lambda i, s: (s[0] * BQ + i, 0, 0, 0)),
            ],
            out_specs=pl.BlockSpec((b, hd), lambda i, s: (0, 0)),
            scratch_shapes=[
                pltpu.VMEM((BQ, hd), jnp.float32),
                pltpu.VMEM((2, h, BQ), jnp.float32),
                pltpu.VMEM((BQ, hd), jnp.float32),
                pltpu.VMEM((2, h, BQ), jnp.float32),
                pltpu.SemaphoreType.DMA((2,)),
                pltpu.SemaphoreType.DMA((2,)),
                pltpu.SemaphoreType.DMA((3,)),
                pltpu.SemaphoreType.DMA((3,)),
            ],
        ),
        out_shape=jax.ShapeDtypeStruct((b, hd), jnp.float32),
        compiler_params=pltpu.CompilerParams(
            vmem_limit_bytes=96 * 1024 * 1024,
            collective_id=0,
            has_side_effects=True,
        ),
    )(qidx_arr, q2, kt, vt)
    return out.reshape(b, sq, h, d)
